# baseline (speedup 1.0000x reference)
import functools

import jax
import jax.numpy as jnp
from jax import lax
from jax.experimental import pallas as pl
from jax.experimental.pallas import tpu as pltpu
from jax.experimental.pallas import tpu_sc as plsc

N_NODES = 10000
N_EDGES = 320000
D = 128
EPS = 1e-5

NC = 2
NS = 16
NW = NC * NS
E_PER_TILE = N_EDGES // NW
CHUNK = 64
N_CHUNKS = E_PER_TILE // CHUNK
REM = E_PER_TILE - N_CHUNKS * CHUNK
NBUF = 4
NR = 12
GLEAD = 3
IPF = 6
ROWS_PER_TILE = 640
LAST_ROWS = N_NODES - (NS - 1) * ROWS_PER_TILE


def _agg_body(h_hbm, src_hbm, dst_hbm, zero_hbm, out_hbm,
              idx2, idx_e, rows, sems, acc):
    c = lax.axis_index("c")
    s = lax.axis_index("s")
    wid = c * NS + s

    isems, gsems, ssems = sems
    def start_idx(i, b10):
        base = wid * E_PER_TILE + i * CHUNK
        pltpu.async_copy(src_hbm.at[pl.ds(base, CHUNK)],
                         idx2.at[b10, 0], isems[b10])
        pltpu.async_copy(dst_hbm.at[pl.ds(base, CHUNK)],
                         idx2.at[b10, 1], isems[b10])

    def wait_idx(b10):
        pltpu.make_async_copy(src_hbm.at[pl.ds(0, CHUNK)], idx2.at[b10, 0],
                              isems[b10]).wait()
        pltpu.make_async_copy(dst_hbm.at[pl.ds(0, CHUNK)], idx2.at[b10, 1],
                              isems[b10]).wait()

    def start_gather(b, b10):
        pltpu.async_copy(h_hbm.at[idx2.at[b10, 0]], rows.at[b], gsems[b])

    def wait_gather(b, b10):
        pltpu.make_async_copy(h_hbm.at[idx2.at[b10, 0]], rows.at[b],
                              gsems[b]).wait()

    def start_scatter(b, b10):
        pltpu.async_copy(rows.at[b], acc.at[idx2.at[b10, 1]], ssems[b],
                         add=True)

    def wait_scatter(b):
        pltpu.make_async_copy(rows.at[b], acc.at[pl.ds(0, CHUNK)],
                              ssems[b]).wait()

    for j in range(IPF):
        start_idx(j, j)
    for j in range(GLEAD):
        wait_idx(j)
        start_gather(j % NBUF, j)

    @pl.when(s < NS - 1)
    def _():
        pltpu.sync_copy(zero_hbm,
                        acc.at[pl.ds(s * ROWS_PER_TILE, ROWS_PER_TILE)])

    @pl.when(s == NS - 1)
    def _():
        pltpu.sync_copy(zero_hbm.at[pl.ds(0, LAST_ROWS)],
                        acc.at[pl.ds((NS - 1) * ROWS_PER_TILE, LAST_ROWS)])

    plsc.subcore_barrier()

    def ring_pass(k, carry):
        for u in range(NR):
            t = k * NR + u
            b = u % NBUF
            q = u % NR
            wait_gather(b, q)
            start_scatter(b, q)

            b2 = (u + GLEAD) % NBUF
            q2 = (u + GLEAD) % NR

            @pl.when(t + GLEAD < N_CHUNKS)
            def _():
                @pl.when(t >= 1)
                def _():
                    wait_scatter(b2)
                wait_idx(q2)
                start_gather(b2, q2)

            q6 = (u + IPF) % NR

            @pl.when(t + IPF < N_CHUNKS)
            def _():
                start_idx(t + IPF, q6)
        return carry

    lax.fori_loop(0, N_CHUNKS // NR, ring_pass, 0)

    for b in range(NBUF):
        wait_scatter(b)

    ebase = wid * E_PER_TILE + N_CHUNKS * CHUNK
    pltpu.async_copy(src_hbm.at[pl.ds(ebase, REM)], idx_e.at[0], isems[0])
    pltpu.async_copy(dst_hbm.at[pl.ds(ebase, REM)], idx_e.at[1], isems[0])
    pltpu.make_async_copy(src_hbm.at[pl.ds(0, REM)], idx_e.at[0],
                          isems[0]).wait()
    pltpu.make_async_copy(dst_hbm.at[pl.ds(0, REM)], idx_e.at[1],
                          isems[0]).wait()
    pltpu.async_copy(h_hbm.at[idx_e.at[0]], rows.at[0, pl.ds(0, REM)],
                     gsems[0]).wait()
    pltpu.async_copy(rows.at[0, pl.ds(0, REM)], acc.at[idx_e.at[1]],
                     ssems[0], add=True)
    pltpu.make_async_copy(rows.at[0, pl.ds(0, REM)], acc.at[pl.ds(0, REM)],
                          ssems[0]).wait()
    plsc.subcore_barrier()

    @pl.when(s < NS - 1)
    def _():
        pltpu.sync_copy(acc.at[pl.ds(s * ROWS_PER_TILE, ROWS_PER_TILE)],
                        out_hbm.at[c, pl.ds(s * ROWS_PER_TILE, ROWS_PER_TILE)])

    @pl.when(s == NS - 1)
    def _():
        pltpu.sync_copy(
            acc.at[pl.ds((NS - 1) * ROWS_PER_TILE, LAST_ROWS)],
            out_hbm.at[c, pl.ds((NS - 1) * ROWS_PER_TILE, LAST_ROWS)])


@functools.cache
def _agg_call():
    return pl.kernel(
        _agg_body,
        out_type=jax.ShapeDtypeStruct((NC, N_NODES, D), jnp.float32),
        mesh=plsc.VectorSubcoreMesh(core_axis_name="c", subcore_axis_name="s",
                                    num_cores=NC, num_subcores=NS),
        scratch_types=[
            pltpu.VMEM((NR, 2, CHUNK), jnp.int32),
            pltpu.VMEM((2, REM), jnp.int32),
            pltpu.VMEM((NBUF, CHUNK, D), jnp.float32),
            ([pltpu.SemaphoreType.DMA] * NR,
             [pltpu.SemaphoreType.DMA] * NBUF,
             [pltpu.SemaphoreType.DMA] * NBUF),
            pltpu.VMEM_SHARED((N_NODES, D), jnp.float32),
        ],
    )


def _scale_body(x_ref, n_ref, e_ref, o_ref, osrc_ref, odst_ref, oz_ref):
    o_ref[...] = x_ref[...] * n_ref[...]
    osrc_ref[...] = e_ref[0].reshape(N_EDGES // D, D)
    odst_ref[...] = e_ref[1].reshape(N_EDGES // D, D)
    oz_ref[...] = jnp.zeros_like(oz_ref)


def _mid_body(p_ref, innorm_ref, w_ref, b_ref, g_ref, be_ref, onorm_ref, o_ref):
    agg = (p_ref[0] + p_ref[1]) * innorm_ref[...]
    t = jnp.dot(agg, w_ref[...], preferred_element_type=jnp.float32) + b_ref[...]
    mu = jnp.mean(t, axis=-1, keepdims=True)
    var = jnp.mean((t - mu) ** 2, axis=-1, keepdims=True)
    t = (t - mu) * lax.rsqrt(var + EPS) * g_ref[...] + be_ref[...]
    t = jnp.maximum(t, 0.0)
    o_ref[...] = t * onorm_ref[...]


def _final_body(p_ref, innorm_ref, w_ref, b_ref, o_ref):
    agg = (p_ref[0] + p_ref[1]) * innorm_ref[...]
    o_ref[...] = jnp.dot(agg, w_ref[...],
                         preferred_element_type=jnp.float32) + b_ref[...]


_scale_call = pl.pallas_call(
    _scale_body,
    out_shape=(
        jax.ShapeDtypeStruct((N_NODES, D), jnp.float32),
        jax.ShapeDtypeStruct((N_EDGES // D, D), jnp.int32),
        jax.ShapeDtypeStruct((N_EDGES // D, D), jnp.int32),
        jax.ShapeDtypeStruct((ROWS_PER_TILE, D), jnp.float32),
    ),
)

_MB = N_NODES // 2
_mid_call = pl.pallas_call(
    _mid_body,
    grid=(2,),
    in_specs=[
        pl.BlockSpec((NC, _MB, D), lambda i: (0, i, 0)),
        pl.BlockSpec((_MB, 1), lambda i: (i, 0)),
        pl.BlockSpec((D, D), lambda i: (0, 0)),
        pl.BlockSpec((1, D), lambda i: (0, 0)),
        pl.BlockSpec((1, D), lambda i: (0, 0)),
        pl.BlockSpec((1, D), lambda i: (0, 0)),
        pl.BlockSpec((_MB, 1), lambda i: (i, 0)),
    ],
    out_specs=pl.BlockSpec((_MB, D), lambda i: (i, 0)),
    out_shape=jax.ShapeDtypeStruct((N_NODES, D), jnp.float32),
)

_final_call = pl.pallas_call(
    _final_body,
    grid=(2,),
    in_specs=[
        pl.BlockSpec((NC, _MB, D), lambda i: (0, i, 0)),
        pl.BlockSpec((_MB, 1), lambda i: (i, 0)),
        pl.BlockSpec((D, D), lambda i: (0, 0)),
        pl.BlockSpec((1, D), lambda i: (0, 0)),
    ],
    out_specs=pl.BlockSpec((_MB, D), lambda i: (i, 0)),
    out_shape=jax.ShapeDtypeStruct((N_NODES, D), jnp.float32),
)


@jax.jit
def kernel(feat, edge_index, in_norm, out_norm, W0, b0, W1, b1, gamma0, beta0):
    eidx = edge_index.astype(jnp.int32)
    b0r = b0.reshape(1, D)
    b1r = b1.reshape(1, D)
    g0r = gamma0.reshape(1, D)
    be0r = beta0.reshape(1, D)

    agg = _agg_call()
    h0, src2, dst2, zero = _scale_call(feat, out_norm, eidx)
    src = src2.reshape(N_EDGES)
    dst = dst2.reshape(N_EDGES)
    p0 = agg(h0, src, dst, zero)
    h1 = _mid_call(p0, in_norm, W0, b0r, g0r, be0r, out_norm)
    p1 = agg(h1, src, dst, zero)
    return _final_call(p1, in_norm, W1, b1r)


# speedup vs baseline: 1.0276x; 1.0276x over previous
import functools

import jax
import jax.numpy as jnp
from jax import lax
from jax.experimental import pallas as pl
from jax.experimental.pallas import tpu as pltpu
from jax.experimental.pallas import tpu_sc as plsc

N_NODES = 10000
N_EDGES = 320000
D = 128
EPS = 1e-5

NC = 2
NS = 16
NW = NC * NS
E_PER_TILE = N_EDGES // NW
CHUNK = 40
N_CHUNKS = E_PER_TILE // CHUNK
NBUF = 5
GLEAD = 4
N_PAD = 10240
ROWS_PER_TILE = N_PAD // NS


def _agg_body(h_hbm, src_hbm, dst_hbm, zero_hbm, out_hbm,
              idx2, rows, sems, acc):
    c = lax.axis_index("c")
    s = lax.axis_index("s")
    wid = c * NS + s

    isems, gsems, ssems = sems
    NR = 2 * NBUF

    def start_idx(i, b10):
        base = wid * E_PER_TILE + i * CHUNK
        pltpu.async_copy(src_hbm.at[pl.ds(base, CHUNK)],
                         idx2.at[b10, 0], isems[b10])
        pltpu.async_copy(dst_hbm.at[pl.ds(base, CHUNK)],
                         idx2.at[b10, 1], isems[b10])

    def wait_idx(b10):
        pltpu.make_async_copy(src_hbm.at[pl.ds(0, CHUNK)], idx2.at[b10, 0],
                              isems[b10]).wait()
        pltpu.make_async_copy(dst_hbm.at[pl.ds(0, CHUNK)], idx2.at[b10, 1],
                              isems[b10]).wait()

    def start_gather(b, b10):
        pltpu.async_copy(h_hbm.at[idx2.at[b10, 0]], rows.at[b], gsems[b])

    def wait_gather(b, b10):
        pltpu.make_async_copy(h_hbm.at[idx2.at[b10, 0]], rows.at[b],
                              gsems[b]).wait()

    def start_scatter(b, b10):
        pltpu.async_copy(rows.at[b], acc.at[idx2.at[b10, 1]], ssems[b],
                         add=True)

    def wait_scatter(b):
        pltpu.make_async_copy(rows.at[b], acc.at[pl.ds(0, CHUNK)],
                              ssems[b]).wait()

    for b in range(NBUF):
        start_idx(b, b)
    for b in range(GLEAD):
        wait_idx(b)
        start_gather(b, b)

    pltpu.sync_copy(zero_hbm,
                    acc.at[pl.ds(s * ROWS_PER_TILE, ROWS_PER_TILE)])
    plsc.subcore_barrier()

    def ring_pass(k2, carry):
        for kk in range(2):
            i0 = (k2 * 2 + kk) * NBUF
            for b in range(NBUF):
                i = i0 + b
                b10 = kk * NBUF + b
                wait_gather(b, b10)
                start_scatter(b, b10)

                nxt_i = i + NBUF
                nxt_b10 = (b10 + NBUF) % NR

                @pl.when(nxt_i < N_CHUNKS)
                def _():
                    start_idx(nxt_i, nxt_b10)

                b2 = (b + GLEAD) % NBUF
                g10 = (b10 + GLEAD) % NR

                @pl.when(i + GLEAD < N_CHUNKS)
                def _():
                    @pl.when(i >= NBUF - GLEAD)
                    def _():
                        wait_scatter(b2)
                    wait_idx(g10)
                    start_gather(b2, g10)
        return carry

    lax.fori_loop(0, N_CHUNKS // (2 * NBUF), ring_pass, 0)

    for b in range(NBUF):
        wait_scatter(b)
    plsc.subcore_barrier()

    @pl.when(s < NS - 1)
    def _():
        pltpu.sync_copy(acc.at[pl.ds(s * ROWS_PER_TILE, ROWS_PER_TILE)],
                        out_hbm.at[c, pl.ds(s * ROWS_PER_TILE, ROWS_PER_TILE)])

    @pl.when(s == NS - 1)
    def _():
        last = N_NODES - (NS - 1) * ROWS_PER_TILE
        pltpu.sync_copy(acc.at[pl.ds((NS - 1) * ROWS_PER_TILE, last)],
                        out_hbm.at[c, pl.ds((NS - 1) * ROWS_PER_TILE, last)])


@functools.cache
def _agg_call():
    return pl.kernel(
        _agg_body,
        out_type=jax.ShapeDtypeStruct((NC, N_NODES, D), jnp.float32),
        mesh=plsc.VectorSubcoreMesh(core_axis_name="c", subcore_axis_name="s",
                                    num_cores=NC, num_subcores=NS),
        scratch_types=[
            pltpu.VMEM((2 * NBUF, 2, CHUNK), jnp.int32),
            pltpu.VMEM((NBUF, CHUNK, D), jnp.float32),
            ([pltpu.SemaphoreType.DMA] * (2 * NBUF),
             [pltpu.SemaphoreType.DMA] * NBUF,
             [pltpu.SemaphoreType.DMA] * NBUF),
            pltpu.VMEM_SHARED((N_PAD, D), jnp.float32),
        ],
    )


def _scale_body(x_ref, n_ref, e_ref, o_ref, osrc_ref, odst_ref, oz_ref):
    o_ref[...] = x_ref[...] * n_ref[...]
    osrc_ref[...] = e_ref[0].reshape(N_EDGES // D, D)
    odst_ref[...] = e_ref[1].reshape(N_EDGES // D, D)
    oz_ref[...] = jnp.zeros_like(oz_ref)


def _mid_body(p_ref, innorm_ref, w_ref, b_ref, g_ref, be_ref, onorm_ref, o_ref):
    agg = (p_ref[0] + p_ref[1]) * innorm_ref[...]
    t = jnp.dot(agg, w_ref[...], preferred_element_type=jnp.float32) + b_ref[...]
    mu = jnp.mean(t, axis=-1, keepdims=True)
    var = jnp.mean((t - mu) ** 2, axis=-1, keepdims=True)
    t = (t - mu) * lax.rsqrt(var + EPS) * g_ref[...] + be_ref[...]
    t = jnp.maximum(t, 0.0)
    o_ref[...] = t * onorm_ref[...]


def _final_body(p_ref, innorm_ref, w_ref, b_ref, o_ref):
    agg = (p_ref[0] + p_ref[1]) * innorm_ref[...]
    o_ref[...] = jnp.dot(agg, w_ref[...],
                         preferred_element_type=jnp.float32) + b_ref[...]


_scale_call = pl.pallas_call(
    _scale_body,
    out_shape=(
        jax.ShapeDtypeStruct((N_NODES, D), jnp.float32),
        jax.ShapeDtypeStruct((N_EDGES // D, D), jnp.int32),
        jax.ShapeDtypeStruct((N_EDGES // D, D), jnp.int32),
        jax.ShapeDtypeStruct((ROWS_PER_TILE, D), jnp.float32),
    ),
)

_MB = N_NODES // 2
_mid_call = pl.pallas_call(
    _mid_body,
    grid=(2,),
    in_specs=[
        pl.BlockSpec((NC, _MB, D), lambda i: (0, i, 0)),
        pl.BlockSpec((_MB, 1), lambda i: (i, 0)),
        pl.BlockSpec((D, D), lambda i: (0, 0)),
        pl.BlockSpec((1, D), lambda i: (0, 0)),
        pl.BlockSpec((1, D), lambda i: (0, 0)),
        pl.BlockSpec((1, D), lambda i: (0, 0)),
        pl.BlockSpec((_MB, 1), lambda i: (i, 0)),
    ],
    out_specs=pl.BlockSpec((_MB, D), lambda i: (i, 0)),
    out_shape=jax.ShapeDtypeStruct((N_NODES, D), jnp.float32),
)

_final_call = pl.pallas_call(
    _final_body,
    grid=(2,),
    in_specs=[
        pl.BlockSpec((NC, _MB, D), lambda i: (0, i, 0)),
        pl.BlockSpec((_MB, 1), lambda i: (i, 0)),
        pl.BlockSpec((D, D), lambda i: (0, 0)),
        pl.BlockSpec((1, D), lambda i: (0, 0)),
    ],
    out_specs=pl.BlockSpec((_MB, D), lambda i: (i, 0)),
    out_shape=jax.ShapeDtypeStruct((N_NODES, D), jnp.float32),
)


@jax.jit
def kernel(feat, edge_index, in_norm, out_norm, W0, b0, W1, b1, gamma0, beta0):
    eidx = edge_index.astype(jnp.int32)
    b0r = b0.reshape(1, D)
    b1r = b1.reshape(1, D)
    g0r = gamma0.reshape(1, D)
    be0r = beta0.reshape(1, D)

    agg = _agg_call()
    h0, src2, dst2, zero = _scale_call(feat, out_norm, eidx)
    src = src2.reshape(N_EDGES)
    dst = dst2.reshape(N_EDGES)
    p0 = agg(h0, src, dst, zero)
    h1 = _mid_call(p0, in_norm, W0, b0r, g0r, be0r, out_norm)
    p1 = agg(h1, src, dst, zero)
    return _final_call(p1, in_norm, W1, b1r)
